# Initial kernel scaffold; baseline (speedup 1.0000x reference)
import jax
import jax.numpy as jnp
from jax.experimental import pallas as pl


def kernel(token_ids, token_table, pos_table):
    raise NotImplementedError("write your pallas kernel here")


# speedup vs baseline: 1.4434x; 1.4434x over previous
import functools

import jax
import jax.numpy as jnp
from jax import lax
from jax.experimental import pallas as pl
from jax.experimental.pallas import tpu as pltpu
from jax.experimental.pallas import tpu_sc as plsc

LANES = 16


def _build_encoder(batch, seq, vocab, dim):
    info = plsc.get_sparse_core_info()
    nc, ns = info.num_cores, info.num_subcores
    nw = nc * ns
    rows_total = batch * seq
    rows_per_w = rows_total // nw
    seqs_per_w = rows_per_w // seq
    seqs_per_chunk = 16
    chunk = seqs_per_chunk * seq
    n_chunks = seqs_per_w // seqs_per_chunk
    gsz = 128
    n_gather = chunk // gsz

    mesh = plsc.VectorSubcoreMesh(core_axis_name="c", subcore_axis_name="s")

    @functools.partial(
        pl.kernel,
        mesh=mesh,
        compiler_params=pltpu.CompilerParams(use_tc_tiling_on_sc=False),
        out_type=jax.ShapeDtypeStruct((rows_total, dim), jnp.float32),
        scratch_types=[
            pltpu.VMEM((chunk,), jnp.int32),
            pltpu.VMEM((chunk, dim), jnp.float32),
            pltpu.VMEM((seq, dim), jnp.float32),
            pltpu.SemaphoreType.DMA,
        ],
    )
    def enc(ids_hbm, table_hbm, pos_hbm, out_hbm, idx_v, rows_v, pos_v, sem):
        wid = lax.axis_index("s") * nc + lax.axis_index("c")
        pltpu.sync_copy(pos_hbm, pos_v)
        for c in range(n_chunks):
            base = wid * rows_per_w + c * chunk
            pltpu.sync_copy(ids_hbm.at[pl.ds(base, chunk)], idx_v)
            descs = []
            for j in range(n_gather):
                descs.append(
                    pltpu.async_copy(
                        table_hbm.at[idx_v.at[pl.ds(j * gsz, gsz)]],
                        rows_v.at[pl.ds(j * gsz, gsz)],
                        sem,
                    )
                )
            for d in descs:
                d.wait()

            def add_pos(pr, carry):
                for h in range(dim // LANES):
                    pv = pos_v[pr, pl.ds(h * LANES, LANES)]
                    for s in range(seqs_per_chunk):
                        r = s * seq + pr
                        rows_v[r, pl.ds(h * LANES, LANES)] = (
                            rows_v[r, pl.ds(h * LANES, LANES)] + pv
                        )
                return carry

            lax.fori_loop(0, seq, add_pos, 0)
            pltpu.sync_copy(rows_v, out_hbm.at[pl.ds(base, chunk)])

    return enc


def kernel(token_ids, token_table, pos_table):
    batch, seq = token_ids.shape
    vocab, dim = token_table.shape
    enc = _build_encoder(batch, seq, vocab, dim)
    out = enc(token_ids.reshape(-1), token_table, pos_table)
    return out.reshape(batch, seq, dim)


# speedup vs baseline: 1.4905x; 1.0326x over previous
import functools

import jax
import jax.numpy as jnp
from jax import lax
from jax.experimental import pallas as pl
from jax.experimental.pallas import tpu as pltpu
from jax.experimental.pallas import tpu_sc as plsc

LANES = 16


def _build_encoder(batch, seq, vocab, dim):
    info = plsc.get_sparse_core_info()
    nc, ns = info.num_cores, info.num_subcores
    nw = nc * ns
    rows_total = batch * seq
    rows_per_w = rows_total // nw
    seqs_per_w = rows_per_w // seq
    seqs_per_chunk = 8
    chunk = seqs_per_chunk * seq
    n_chunks = seqs_per_w // seqs_per_chunk
    gsz = 80
    n_gather = chunk // gsz

    mesh = plsc.VectorSubcoreMesh(core_axis_name="c", subcore_axis_name="s")

    @functools.partial(
        pl.kernel,
        mesh=mesh,
        compiler_params=pltpu.CompilerParams(use_tc_tiling_on_sc=False),
        out_type=jax.ShapeDtypeStruct((rows_total, dim), jnp.float32),
        scratch_types=[
            pltpu.VMEM((chunk,), jnp.int32),
            pltpu.VMEM((chunk,), jnp.int32),
            pltpu.VMEM((chunk, dim), jnp.float32),
            pltpu.VMEM((chunk, dim), jnp.float32),
            pltpu.VMEM((seq, dim), jnp.float32),
            pltpu.SemaphoreType.DMA,
            pltpu.SemaphoreType.DMA,
            pltpu.SemaphoreType.DMA,
            pltpu.SemaphoreType.DMA,
            pltpu.SemaphoreType.DMA,
            pltpu.SemaphoreType.DMA,
        ],
    )
    def enc(ids_hbm, table_hbm, pos_hbm, out_hbm,
            idx_v0, idx_v1, rows_v0, rows_v1, pos_v,
            sg0, sg1, si0, si1, so0, so1):
        idx_v = (idx_v0, idx_v1)
        rows_v = (rows_v0, rows_v1)
        sg = (sg0, sg1)
        si = (si0, si1)
        so = (so0, so1)
        wid = lax.axis_index("s") * nc + lax.axis_index("c")
        base0 = wid * rows_per_w
        pltpu.sync_copy(pos_hbm, pos_v)

        def fire_gathers(b):
            return [
                pltpu.async_copy(
                    table_hbm.at[idx_v[b].at[pl.ds(j * gsz, gsz)]],
                    rows_v[b].at[pl.ds(j * gsz, gsz)],
                    sg[b],
                )
                for j in range(n_gather)
            ]

        def add_pos(b):
            def body(pr, carry):
                for h in range(dim // LANES):
                    pv = pos_v[pr, pl.ds(h * LANES, LANES)]
                    for s in range(seqs_per_chunk):
                        plsc.addupdate(
                            rows_v[b].at[s * seq + pr, pl.ds(h * LANES, LANES)],
                            pv,
                        )
                return carry
            lax.fori_loop(0, seq, body, 0)

        pltpu.sync_copy(ids_hbm.at[pl.ds(base0, chunk)], idx_v[0])
        gd = [fire_gathers(0), None]
        id_desc = [None, None]
        od = [None, None]
        if n_chunks > 1:
            id_desc[1] = pltpu.async_copy(
                ids_hbm.at[pl.ds(base0 + chunk, chunk)], idx_v[1], si[1])

        for c in range(n_chunks):
            b = c & 1
            nb = 1 - b
            for d in gd[b]:
                d.wait()
            if c + 1 < n_chunks:
                id_desc[nb].wait()
                if od[nb] is not None:
                    od[nb].wait()
                gd[nb] = fire_gathers(nb)
            if c + 2 < n_chunks:
                id_desc[b] = pltpu.async_copy(
                    ids_hbm.at[pl.ds(base0 + (c + 2) * chunk, chunk)],
                    idx_v[b], si[b])
            add_pos(b)
            od[b] = pltpu.async_copy(
                rows_v[b], out_hbm.at[pl.ds(base0 + c * chunk, chunk)], so[b])

        od[(n_chunks - 2) & 1].wait()
        od[(n_chunks - 1) & 1].wait()

    return enc


def kernel(token_ids, token_table, pos_table):
    batch, seq = token_ids.shape
    vocab, dim = token_table.shape
    enc = _build_encoder(batch, seq, vocab, dim)
    out = enc(token_ids.reshape(-1), token_table, pos_table)
    return out.reshape(batch, seq, dim)
